# baseline (speedup 1.0000x reference)
import functools

import numpy as np
import jax
import jax.numpy as jnp
from jax import lax
from jax.experimental import pallas as pl
from jax.experimental.pallas import tpu as pltpu
from jax.experimental.pallas import tpu_sc as plsc

N_LEVELS = 16
F = 2
LOG2_T = 19
T = 1 << LOG2_T
BASE = 16
FINEST = 512
OUT_DIM = 16
N_POINTS = 524288

_b = np.exp((np.log(FINEST) - np.log(BASE)) / (N_LEVELS - 1))
RES = [int(np.floor(BASE * (_b ** l))) for l in range(N_LEVELS)]
P1 = int(np.uint32(2654435761).astype(np.int32))
P2 = int(np.uint32(805459861).astype(np.int32))

NC, NS = 2, 16
NW = NC * NS
PTS = N_POINTS // NW
P = 256
G = P // 16
NCHUNK = PTS // P


LVL = T // 4


def _run_level(l, base0, tab_ref, x_hbm, feats_hbm, xr_v, idx_v, lo_v, w_v,
               rows_v, fb_v, res_v, semg, semx, semf, iot, zeros16):
    lb = zeros16 + l
    r_b = plsc.load_gather(res_v, [lb])

    pltpu.async_copy(x_hbm.at[:, pl.ds(base0, P)], xr_v.at[0], semx[0])

    def phase1(j, b):
        @pl.when(j + 1 < NCHUNK)
        def _():
            pltpu.async_copy(x_hbm.at[:, pl.ds(base0 + (j + 1) * P, P)],
                             xr_v.at[1 - b], semx[1 - b])
        pltpu.make_async_copy(x_hbm.at[:, pl.ds(base0, P)],
                              xr_v.at[b], semx[b]).wait()

        def grp_body(g, c3):
            o = g * 16

            def nrm(d):
                v = xr_v[b, d, pl.ds(o, 16)]
                v = jnp.minimum(jnp.maximum(v * 0.5 + 0.5, 0.0), 1.0)
                return v * r_b
            pos0 = nrm(0)
            pos1 = nrm(1)
            pos2 = nrm(2)
            i0 = pos0.astype(jnp.int32)
            i1 = pos1.astype(jnp.int32)
            i2 = pos2.astype(jnp.int32)
            w0 = pos0 - i0.astype(jnp.float32)
            w1 = pos1 - i1.astype(jnp.float32)
            w2 = pos2 - i2.astype(jnp.float32)
            hx = (i0, i0 + 1)
            hy0 = i1 * P1
            hy = (hy0, hy0 + P1)
            hz0 = i2 * P2
            hz = (hz0, hz0 + P2)
            u0 = 1.0 - w0
            u1 = 1.0 - w1
            u2 = 1.0 - w2
            wx = (u0, w0)
            cyz = (u1 * u2, u1 * w2, w1 * u2, w1 * w2)
            eyz = (hy[0] ^ hz[0], hy[0] ^ hz[1], hy[1] ^ hz[0], hy[1] ^ hz[1])
            for c in range(8):
                bi, bj, bk = (c >> 2) & 1, (c >> 1) & 1, c & 1
                h = hx[bi] ^ eyz[bj * 2 + bk]
                idxc = h & (T - 1)
                wc = wx[bi] * cyz[bj * 2 + bk]
                idx_v[b, g, pl.ds(c * 16, 16)] = lax.shift_right_logical(idxc, 2)
                lo_v[b, pl.ds(g * 128 + c * 16, 16)] = (idxc & 3) * 2
                w_v[b, pl.ds(g * 128 + c * 16, 16)] = wc
            pltpu.async_copy(tab_ref.at[idx_v.at[b].at[g]],
                             rows_v.at[b].at[pl.ds(g * 128, 128)], semg[b])
            return c3

        @plsc.parallel_loop(0, G, unroll=3)
        def _(g):
            grp_body(g, 0)

    def phase2(q, b):
        @pl.when(q >= 2)
        def _():
            pltpu.make_async_copy(
                fb_v.at[b],
                feats_hbm.at[pl.ds(2 * l, 2), pl.ds(base0, P)],
                semf[b]).wait()

        def acc_body(g, c3):
            pltpu.make_async_copy(tab_ref.at[idx_v.at[b].at[g]],
                                  rows_v.at[b].at[pl.ds(g * 128, 128)],
                                  semg[b]).wait()
            acc0 = jnp.zeros((16,), jnp.float32)
            acc1 = jnp.zeros((16,), jnp.float32)
            for c in range(8):
                lanes = g * 128 + c * 16 + iot
                lo = lo_v[b, pl.ds(g * 128 + c * 16, 16)]
                f0 = plsc.load_gather(rows_v.at[b], [lanes, lo])
                f1 = plsc.load_gather(rows_v.at[b], [lanes, lo + 1])
                wc = w_v[b, pl.ds(g * 128 + c * 16, 16)]
                acc0 = acc0 + f0 * wc
                acc1 = acc1 + f1 * wc
            fb_v[b, 0, pl.ds(g * 16, 16)] = acc0
            fb_v[b, 1, pl.ds(g * 16, 16)] = acc1
            return c3

        @plsc.parallel_loop(0, G, unroll=3)
        def _(g):
            acc_body(g, 0)
        pltpu.async_copy(fb_v.at[b],
                         feats_hbm.at[pl.ds(2 * l, 2),
                                      pl.ds(base0 + q * P, P)], semf[b])

    def loop_body(j, carry):
        for par in (0, 1):
            @pl.when(jnp.logical_and(j < NCHUNK, j % 2 == par))
            def _(par=par):
                phase1(j, par)
        q = j - 1
        for par in (0, 1):
            @pl.when(jnp.logical_and(j > 0, q % 2 == par))
            def _(par=par):
                phase2(q, par)
        return carry
    lax.fori_loop(0, NCHUNK + 1, loop_body, 0)

    for par in (0, 1):
        pltpu.make_async_copy(
            fb_v.at[par],
            feats_hbm.at[pl.ds(2 * l, 2), pl.ds(base0, P)], semf[par]).wait()


def _encode_body(x_hbm, tab_hbm, res_hbm, feats_hbm,
                 xr_v, idx_v, lo_v, w_v, rows_v, fb_v, res_v,
                 tabA, semg0, semg1, semx0, semx1, semf0, semf1):
    cid = lax.axis_index("c")
    sid = lax.axis_index("s")
    wid = sid * NC + cid
    base0 = wid * PTS
    pltpu.sync_copy(res_hbm, res_v)
    iot = lax.iota(jnp.int32, 16)
    zeros16 = jnp.zeros((16,), jnp.int32)

    SL = LVL // NS

    def level_body(l, carry):
        pltpu.sync_copy(tab_hbm.at[pl.ds(l * LVL + sid * SL, SL)],
                        tabA.at[pl.ds(sid * SL, SL)])
        plsc.subcore_barrier()
        _run_level(l, base0, tabA, x_hbm, feats_hbm, xr_v, idx_v, lo_v,
                   w_v, rows_v, fb_v, res_v, (semg0, semg1),
                   (semx0, semx1), (semf0, semf1), iot, zeros16)
        plsc.subcore_barrier()
        return carry
    lax.fori_loop(0, N_LEVELS, level_body, 0)


def _make_encode():
    mesh = plsc.VectorSubcoreMesh(core_axis_name="c", subcore_axis_name="s",
                                  num_cores=NC, num_subcores=NS)
    return pl.kernel(
        _encode_body,
        out_type=jax.ShapeDtypeStruct((2 * N_LEVELS, N_POINTS), jnp.float32),
        mesh=mesh,
        scratch_types=[
            pltpu.VMEM((2, 3, P), jnp.float32),
            pltpu.VMEM((2, G, 128), jnp.int32),
            pltpu.VMEM((2, 8 * P), jnp.int32),
            pltpu.VMEM((2, 8 * P), jnp.float32),
            pltpu.VMEM((2, 8 * P, 8), jnp.float32),
            pltpu.VMEM((2, 2, P), jnp.float32),
            pltpu.VMEM((16,), jnp.float32),
            pltpu.VMEM_SHARED((LVL, 8), jnp.float32),
            pltpu.SemaphoreType.DMA,
            pltpu.SemaphoreType.DMA,
            pltpu.SemaphoreType.DMA,
            pltpu.SemaphoreType.DMA,
            pltpu.SemaphoreType.DMA,
            pltpu.SemaphoreType.DMA,
        ],
        compiler_params=pltpu.CompilerParams(needs_layout_passes=False,
                                             use_tc_tiling_on_sc=False),
    )


def _tr_body(i_ref, p_ref, o_ref):
    a = i_ref[...]
    a = a.reshape(a.shape[0] // 2, 256)
    b = lax.dot_general(a, p_ref[...], (((1,), (0,)), ((), ())),
                        preferred_element_type=jnp.float32,
                        precision=lax.Precision.HIGHEST)
    o_ref[...] = b.reshape(a.shape[0] * 2, 128)


BT = 1024
TR_ROWS = N_LEVELS * T // 64

_PERM = np.zeros((256, 256), dtype=np.float32)
for _j in range(256):
    _PERM[(_j % 2) * 128 + _j // 2, _j] = 1.0


def _transpose_tables(gtab):
    return pl.pallas_call(
        _tr_body,
        grid=(TR_ROWS // BT,),
        in_specs=[pl.BlockSpec((BT, 128), lambda i: (i, 0)),
                  pl.BlockSpec((256, 256), lambda i: (0, 0))],
        out_specs=pl.BlockSpec((BT, 128), lambda i: (i, 0)),
        out_shape=jax.ShapeDtypeStruct((TR_ROWS, 128), jnp.float32),
    )(gtab, jnp.asarray(_PERM))


def _mlp_body(f_ref, w1_ref, b1_ref, w2_ref, b2_ref, w3_ref, b3_ref, o_ref):
    a3 = f_ref[...]
    a = a3.reshape(2 * N_LEVELS, PB)
    h = lax.dot_general(w1_ref[...], a, (((0,), (0,)), ((), ())),
                        preferred_element_type=jnp.float32)
    h = jnp.maximum(h + b1_ref[...], 0.0)
    h = lax.dot_general(w2_ref[...], h, (((0,), (0,)), ((), ())),
                        preferred_element_type=jnp.float32)
    h = jnp.maximum(h + b2_ref[...], 0.0)
    o = lax.dot_general(w3_ref[...], h, (((0,), (0,)), ((), ())),
                        preferred_element_type=jnp.float32)
    o_ref[...] = o + b3_ref[...]


PB = 2048


def _mlp(feats3, W1, b1, W2, b2, W3, b3):
    return pl.pallas_call(
        _mlp_body,
        grid=(N_POINTS // PB,),
        in_specs=[
            pl.BlockSpec((2 * N_LEVELS, PB // 128, 128), lambda i: (0, i, 0)),
            pl.BlockSpec((2 * N_LEVELS, 64), lambda i: (0, 0)),
            pl.BlockSpec((64, 1), lambda i: (0, 0)),
            pl.BlockSpec((64, 64), lambda i: (0, 0)),
            pl.BlockSpec((64, 1), lambda i: (0, 0)),
            pl.BlockSpec((64, OUT_DIM), lambda i: (0, 0)),
            pl.BlockSpec((OUT_DIM, 1), lambda i: (0, 0)),
        ],
        out_specs=pl.BlockSpec((OUT_DIM, PB), lambda i: (0, i)),
        out_shape=jax.ShapeDtypeStruct((OUT_DIM, N_POINTS), jnp.float32),
    )(feats3, W1, b1.reshape(64, 1), W2, b2.reshape(64, 1),
      W3, b3.reshape(OUT_DIM, 1))


@jax.jit
def _impl(x, tables, W1, b1, W2, b2, W3, b3):
    xt = x.T
    gtab = tables.reshape(N_LEVELS, T // 128, 128, 2)
    gtab = gtab.transpose(0, 1, 3, 2).reshape(TR_ROWS, 128)
    tab8 = _transpose_tables(gtab).reshape(N_LEVELS * T // 4, 4 * F)
    res = jnp.asarray(np.array(RES, np.float32))
    feats = _make_encode()(xt, tab8, res)
    feats3 = feats.reshape(2 * N_LEVELS, N_POINTS // 128, 128)
    out16 = _mlp(feats3, W1, b1, W2, b2, W3, b3)
    return out16.T


def kernel(x, tables, W1, b1, W2, b2, W3, b3):
    return _impl(x, tables, W1, b1, W2, b2, W3, b3)


# speedup vs baseline: 1.4351x; 1.4351x over previous
import functools

import numpy as np
import jax
import jax.numpy as jnp
from jax import lax
from jax.experimental import pallas as pl
from jax.experimental.pallas import tpu as pltpu
from jax.experimental.pallas import tpu_sc as plsc

N_LEVELS = 16
F = 2
LOG2_T = 19
T = 1 << LOG2_T
BASE = 16
FINEST = 512
OUT_DIM = 16
N_POINTS = 524288

_b = np.exp((np.log(FINEST) - np.log(BASE)) / (N_LEVELS - 1))
RES = [int(np.floor(BASE * (_b ** l))) for l in range(N_LEVELS)]
P1 = int(np.uint32(2654435761).astype(np.int32))
P2 = int(np.uint32(805459861).astype(np.int32))

NC, NS = 2, 16
NW = NC * NS
PTS = N_POINTS // NW
P = 256
G = P // 16
NCHUNK = PTS // P


LVL = T // 4


def _run_level(l, base0, tab_ref, x_hbm, feats_hbm, xr_v, idx_v, lo_v, w_v,
               rows_v, fb_v, res_v, semg, semx, semf, iot, zeros16):
    lb = zeros16 + l
    r_b = plsc.load_gather(res_v, [lb])

    pltpu.async_copy(x_hbm.at[:, pl.ds(base0, P)], xr_v.at[0], semx[0])

    def phase1(j, b):
        @pl.when(j + 1 < NCHUNK)
        def _():
            pltpu.async_copy(x_hbm.at[:, pl.ds(base0 + (j + 1) * P, P)],
                             xr_v.at[1 - b], semx[1 - b])
        pltpu.make_async_copy(x_hbm.at[:, pl.ds(base0, P)],
                              xr_v.at[b], semx[b]).wait()

        def grp_body(g, c3):
            o = g * 16

            def nrm(d):
                v = xr_v[b, d, pl.ds(o, 16)]
                v = jnp.minimum(jnp.maximum(v * 0.5 + 0.5, 0.0), 1.0)
                return v * r_b
            pos0 = nrm(0)
            pos1 = nrm(1)
            pos2 = nrm(2)
            i0 = pos0.astype(jnp.int32)
            i1 = pos1.astype(jnp.int32)
            i2 = pos2.astype(jnp.int32)
            w0 = pos0 - i0.astype(jnp.float32)
            w1 = pos1 - i1.astype(jnp.float32)
            w2 = pos2 - i2.astype(jnp.float32)
            hx = (i0, i0 + 1)
            hy0 = i1 * P1
            hy = (hy0, hy0 + P1)
            hz0 = i2 * P2
            hz = (hz0, hz0 + P2)
            u0 = 1.0 - w0
            u1 = 1.0 - w1
            u2 = 1.0 - w2
            wx = (u0, w0)
            cyz = (u1 * u2, u1 * w2, w1 * u2, w1 * w2)
            eyz = (hy[0] ^ hz[0], hy[0] ^ hz[1], hy[1] ^ hz[0], hy[1] ^ hz[1])
            for c in range(8):
                bi, bj, bk = (c >> 2) & 1, (c >> 1) & 1, c & 1
                h = hx[bi] ^ eyz[bj * 2 + bk]
                idxc = h & (T - 1)
                wc = wx[bi] * cyz[bj * 2 + bk]
                idx_v[b, g, pl.ds(c * 16, 16)] = lax.shift_right_logical(idxc, 2)
                lo_v[b, pl.ds(g * 128 + c * 16, 16)] = (idxc & 3) * 2
                w_v[b, pl.ds(g * 128 + c * 16, 16)] = wc
            pltpu.async_copy(tab_ref.at[idx_v.at[b].at[g]],
                             rows_v.at[b].at[pl.ds(g * 128, 128)], semg[b])
            return c3

        @plsc.parallel_loop(0, G, unroll=2)
        def _(g):
            grp_body(g, 0)

    def phase2(q, b):
        @pl.when(q >= 2)
        def _():
            pltpu.make_async_copy(
                fb_v.at[b],
                feats_hbm.at[pl.ds(2 * l, 2), pl.ds(base0, P)],
                semf[b]).wait()

        def acc_body(g, c3):
            pltpu.make_async_copy(tab_ref.at[idx_v.at[b].at[g]],
                                  rows_v.at[b].at[pl.ds(g * 128, 128)],
                                  semg[b]).wait()
            acc0 = jnp.zeros((16,), jnp.float32)
            acc1 = jnp.zeros((16,), jnp.float32)
            for c in range(8):
                lanes = g * 128 + c * 16 + iot
                lo = lo_v[b, pl.ds(g * 128 + c * 16, 16)]
                f0 = plsc.load_gather(rows_v.at[b], [lanes, lo])
                f1 = plsc.load_gather(rows_v.at[b], [lanes, lo + 1])
                wc = w_v[b, pl.ds(g * 128 + c * 16, 16)]
                acc0 = acc0 + f0 * wc
                acc1 = acc1 + f1 * wc
            fb_v[b, 0, pl.ds(g * 16, 16)] = acc0
            fb_v[b, 1, pl.ds(g * 16, 16)] = acc1
            return c3

        @plsc.parallel_loop(0, G, unroll=2)
        def _(g):
            acc_body(g, 0)
        pltpu.async_copy(fb_v.at[b],
                         feats_hbm.at[pl.ds(2 * l, 2),
                                      pl.ds(base0 + q * P, P)], semf[b])

    def loop_body(j, carry):
        for par in (0, 1):
            @pl.when(jnp.logical_and(j < NCHUNK, j % 2 == par))
            def _(par=par):
                phase1(j, par)
        q = j - 1
        for par in (0, 1):
            @pl.when(jnp.logical_and(j > 0, q % 2 == par))
            def _(par=par):
                phase2(q, par)
        return carry
    lax.fori_loop(0, NCHUNK + 1, loop_body, 0)

    for par in (0, 1):
        pltpu.make_async_copy(
            fb_v.at[par],
            feats_hbm.at[pl.ds(2 * l, 2), pl.ds(base0, P)], semf[par]).wait()


def _encode_body(x_hbm, tab_hbm, res_hbm, feats_hbm,
                 xr_v, idx_v, lo_v, w_v, rows_v, fb_v, res_v,
                 tabA, semg0, semg1, semx0, semx1, semf0, semf1):
    cid = lax.axis_index("c")
    sid = lax.axis_index("s")
    wid = sid * NC + cid
    base0 = wid * PTS
    pltpu.sync_copy(res_hbm, res_v)
    iot = lax.iota(jnp.int32, 16)
    zeros16 = jnp.zeros((16,), jnp.int32)

    SL = LVL // NS

    def level_body(l, carry):
        pltpu.sync_copy(tab_hbm.at[pl.ds(l * LVL + sid * SL, SL)],
                        tabA.at[pl.ds(sid * SL, SL)])
        plsc.subcore_barrier()
        _run_level(l, base0, tabA, x_hbm, feats_hbm, xr_v, idx_v, lo_v,
                   w_v, rows_v, fb_v, res_v, (semg0, semg1),
                   (semx0, semx1), (semf0, semf1), iot, zeros16)
        plsc.subcore_barrier()
        return carry
    lax.fori_loop(0, N_LEVELS, level_body, 0)


def _make_encode():
    mesh = plsc.VectorSubcoreMesh(core_axis_name="c", subcore_axis_name="s",
                                  num_cores=NC, num_subcores=NS)
    return pl.kernel(
        _encode_body,
        out_type=jax.ShapeDtypeStruct((2 * N_LEVELS, N_POINTS), jnp.float32),
        mesh=mesh,
        scratch_types=[
            pltpu.VMEM((2, 3, P), jnp.float32),
            pltpu.VMEM((2, G, 128), jnp.int32),
            pltpu.VMEM((2, 8 * P), jnp.int32),
            pltpu.VMEM((2, 8 * P), jnp.float32),
            pltpu.VMEM((2, 8 * P, 8), jnp.float32),
            pltpu.VMEM((2, 2, P), jnp.float32),
            pltpu.VMEM((16,), jnp.float32),
            pltpu.VMEM_SHARED((LVL, 8), jnp.float32),
            pltpu.SemaphoreType.DMA,
            pltpu.SemaphoreType.DMA,
            pltpu.SemaphoreType.DMA,
            pltpu.SemaphoreType.DMA,
            pltpu.SemaphoreType.DMA,
            pltpu.SemaphoreType.DMA,
        ],
        compiler_params=pltpu.CompilerParams(needs_layout_passes=False,
                                             use_tc_tiling_on_sc=False),
    )


def _tr_body(i_ref, p_ref, o_ref):
    a = i_ref[...]
    a = a.reshape(a.shape[0] // 2, 256)
    b = lax.dot_general(a, p_ref[...], (((1,), (0,)), ((), ())),
                        preferred_element_type=jnp.float32,
                        precision=lax.Precision.HIGHEST)
    o_ref[...] = b.reshape(a.shape[0] * 2, 128)


BT = 2048
TR_ROWS = N_LEVELS * T // 64

_PERM = np.zeros((256, 256), dtype=np.float32)
for _j in range(256):
    _PERM[(_j % 2) * 128 + _j // 2, _j] = 1.0


def _transpose_tables(gtab):
    return pl.pallas_call(
        _tr_body,
        grid=(TR_ROWS // BT,),
        in_specs=[pl.BlockSpec((BT, 128), lambda i: (i, 0)),
                  pl.BlockSpec((256, 256), lambda i: (0, 0))],
        out_specs=pl.BlockSpec((BT, 128), lambda i: (i, 0)),
        out_shape=jax.ShapeDtypeStruct((TR_ROWS, 128), jnp.float32),
    )(gtab, jnp.asarray(_PERM))


def _mlp_body(f_ref, w1_ref, b1_ref, w2_ref, b2_ref, w3_ref, b3_ref, o_ref):
    a3 = f_ref[...]
    a = a3.reshape(2 * N_LEVELS, PB)
    h = lax.dot_general(w1_ref[...], a, (((0,), (0,)), ((), ())),
                        preferred_element_type=jnp.float32)
    h = jnp.maximum(h + b1_ref[...], 0.0)
    h = lax.dot_general(w2_ref[...], h, (((0,), (0,)), ((), ())),
                        preferred_element_type=jnp.float32)
    h = jnp.maximum(h + b2_ref[...], 0.0)
    o = lax.dot_general(w3_ref[...], h, (((0,), (0,)), ((), ())),
                        preferred_element_type=jnp.float32)
    o_ref[...] = o + b3_ref[...]


PB = 4096


def _mlp(feats3, W1, b1, W2, b2, W3, b3):
    return pl.pallas_call(
        _mlp_body,
        grid=(N_POINTS // PB,),
        in_specs=[
            pl.BlockSpec((2 * N_LEVELS, PB // 128, 128), lambda i: (0, i, 0)),
            pl.BlockSpec((2 * N_LEVELS, 64), lambda i: (0, 0)),
            pl.BlockSpec((64, 1), lambda i: (0, 0)),
            pl.BlockSpec((64, 64), lambda i: (0, 0)),
            pl.BlockSpec((64, 1), lambda i: (0, 0)),
            pl.BlockSpec((64, OUT_DIM), lambda i: (0, 0)),
            pl.BlockSpec((OUT_DIM, 1), lambda i: (0, 0)),
        ],
        out_specs=pl.BlockSpec((OUT_DIM, PB), lambda i: (0, i)),
        out_shape=jax.ShapeDtypeStruct((OUT_DIM, N_POINTS), jnp.float32),
    )(feats3, W1, b1.reshape(64, 1), W2, b2.reshape(64, 1),
      W3, b3.reshape(OUT_DIM, 1))


@jax.jit
def _impl(x, tables, W1, b1, W2, b2, W3, b3):
    xt = x.T
    gtab = tables.reshape(N_LEVELS, T // 128, 128, 2)
    gtab = gtab.transpose(0, 1, 3, 2).reshape(TR_ROWS, 128)
    tab8 = _transpose_tables(gtab).reshape(N_LEVELS * T // 4, 4 * F)
    res = jnp.asarray(np.array(RES, np.float32))
    feats = _make_encode()(xt, tab8, res)
    feats3 = feats.reshape(2 * N_LEVELS, N_POINTS // 128, 128)
    out16 = _mlp(feats3, W1, b1, W2, b2, W3, b3)
    return out16.T


def kernel(x, tables, W1, b1, W2, b2, W3, b3):
    return _impl(x, tables, W1, b1, W2, b2, W3, b3)


# speedup vs baseline: 1.5155x; 1.0560x over previous
import functools

import numpy as np
import jax
import jax.numpy as jnp
from jax import lax
from jax.experimental import pallas as pl
from jax.experimental.pallas import tpu as pltpu
from jax.experimental.pallas import tpu_sc as plsc

N_LEVELS = 16
F = 2
LOG2_T = 19
T = 1 << LOG2_T
BASE = 16
FINEST = 512
OUT_DIM = 16
N_POINTS = 524288

_b = np.exp((np.log(FINEST) - np.log(BASE)) / (N_LEVELS - 1))
RES = [int(np.floor(BASE * (_b ** l))) for l in range(N_LEVELS)]
P1 = int(np.uint32(2654435761).astype(np.int32))
P2 = int(np.uint32(805459861).astype(np.int32))

NC, NS = 2, 16
NW = NC * NS
PTS = N_POINTS // NW
P = 256
G = P // 16
NCHUNK = PTS // P


LVL = T // 4


def _run_level(l, base0, tab_ref, x_hbm, feats_hbm, xr_v, idx_v, lo_v, w_v,
               rows_v, fb_v, res_v, semg, semx, semf, iot, zeros16):
    lb = zeros16 + l
    r_b = plsc.load_gather(res_v, [lb])

    pltpu.async_copy(x_hbm.at[:, pl.ds(base0, P)], xr_v.at[0], semx[0])

    def phase1(j, b):
        @pl.when(j + 1 < NCHUNK)
        def _():
            pltpu.async_copy(x_hbm.at[:, pl.ds(base0 + (j + 1) * P, P)],
                             xr_v.at[1 - b], semx[1 - b])
        pltpu.make_async_copy(x_hbm.at[:, pl.ds(base0, P)],
                              xr_v.at[b], semx[b]).wait()

        def grp_body(g, c3):
            o = g * 16

            def nrm(d):
                v = xr_v[b, d, pl.ds(o, 16)]
                v = jnp.minimum(jnp.maximum(v * 0.5 + 0.5, 0.0), 1.0)
                return v * r_b
            pos0 = nrm(0)
            pos1 = nrm(1)
            pos2 = nrm(2)
            i0 = pos0.astype(jnp.int32)
            i1 = pos1.astype(jnp.int32)
            i2 = pos2.astype(jnp.int32)
            w0 = pos0 - i0.astype(jnp.float32)
            w1 = pos1 - i1.astype(jnp.float32)
            w2 = pos2 - i2.astype(jnp.float32)
            hx = (i0, i0 + 1)
            hy0 = i1 * P1
            hy = (hy0, hy0 + P1)
            hz0 = i2 * P2
            hz = (hz0, hz0 + P2)
            u0 = 1.0 - w0
            u1 = 1.0 - w1
            u2 = 1.0 - w2
            wx = (u0, w0)
            cyz = (u1 * u2, u1 * w2, w1 * u2, w1 * w2)
            eyz = (hy[0] ^ hz[0], hy[0] ^ hz[1], hy[1] ^ hz[0], hy[1] ^ hz[1])
            for c in range(8):
                bi, bj, bk = (c >> 2) & 1, (c >> 1) & 1, c & 1
                h = hx[bi] ^ eyz[bj * 2 + bk]
                idxc = h & (T - 1)
                wc = wx[bi] * cyz[bj * 2 + bk]
                idx_v[b, g, pl.ds(c * 16, 16)] = lax.shift_right_logical(idxc, 2)
                lo_v[b, pl.ds(g * 128 + c * 16, 16)] = (idxc & 3) * 2
                w_v[b, pl.ds(g * 128 + c * 16, 16)] = wc
            pltpu.async_copy(tab_ref.at[idx_v.at[b].at[g]],
                             rows_v.at[b].at[pl.ds(g * 128, 128)], semg[b])
            return c3

        @plsc.parallel_loop(0, G, unroll=2)
        def _(g):
            grp_body(g, 0)

    def phase2(q, b):
        @pl.when(q >= 2)
        def _():
            pltpu.make_async_copy(
                fb_v.at[b],
                feats_hbm.at[pl.ds(2 * l, 2), pl.ds(base0, P)],
                semf[b]).wait()

        def acc_body(g, c3):
            pltpu.make_async_copy(tab_ref.at[idx_v.at[b].at[g]],
                                  rows_v.at[b].at[pl.ds(g * 128, 128)],
                                  semg[b]).wait()
            acc0 = jnp.zeros((16,), jnp.float32)
            acc1 = jnp.zeros((16,), jnp.float32)
            for c in range(8):
                lanes = g * 128 + c * 16 + iot
                lo = lo_v[b, pl.ds(g * 128 + c * 16, 16)]
                f0 = plsc.load_gather(rows_v.at[b], [lanes, lo])
                f1 = plsc.load_gather(rows_v.at[b], [lanes, lo + 1])
                wc = w_v[b, pl.ds(g * 128 + c * 16, 16)]
                acc0 = acc0 + f0 * wc
                acc1 = acc1 + f1 * wc
            fb_v[b, 0, pl.ds(g * 16, 16)] = acc0
            fb_v[b, 1, pl.ds(g * 16, 16)] = acc1
            return c3

        @plsc.parallel_loop(0, G, unroll=2)
        def _(g):
            acc_body(g, 0)
        pltpu.async_copy(fb_v.at[b],
                         feats_hbm.at[pl.ds(2 * l, 2),
                                      pl.ds(base0 + q * P, P)], semf[b])

    def loop_body(j, carry):
        for par in (0, 1):
            @pl.when(jnp.logical_and(j < NCHUNK, j % 2 == par))
            def _(par=par):
                phase1(j, par)
        q = j - 1
        for par in (0, 1):
            @pl.when(jnp.logical_and(j > 0, q % 2 == par))
            def _(par=par):
                phase2(q, par)
        return carry
    lax.fori_loop(0, NCHUNK + 1, loop_body, 0)

    for par in (0, 1):
        pltpu.make_async_copy(
            fb_v.at[par],
            feats_hbm.at[pl.ds(2 * l, 2), pl.ds(base0, P)], semf[par]).wait()


def _encode_body(x_hbm, tab_hbm, res_hbm, feats_hbm,
                 xr_v, idx_v, lo_v, w_v, rows_v, fb_v, res_v,
                 tabA, semg0, semg1, semx0, semx1, semf0, semf1):
    cid = lax.axis_index("c")
    sid = lax.axis_index("s")
    wid = sid * NC + cid
    base0 = wid * PTS
    pltpu.sync_copy(res_hbm, res_v)
    iot = lax.iota(jnp.int32, 16)
    zeros16 = jnp.zeros((16,), jnp.int32)

    SL = LVL // NS

    def level_body(l, carry):
        pltpu.sync_copy(tab_hbm.at[pl.ds(l * LVL + sid * SL, SL)],
                        tabA.at[pl.ds(sid * SL, SL)])
        plsc.subcore_barrier()
        _run_level(l, base0, tabA, x_hbm, feats_hbm, xr_v, idx_v, lo_v,
                   w_v, rows_v, fb_v, res_v, (semg0, semg1),
                   (semx0, semx1), (semf0, semf1), iot, zeros16)
        plsc.subcore_barrier()
        return carry
    lax.fori_loop(0, N_LEVELS, level_body, 0)


def _make_encode():
    mesh = plsc.VectorSubcoreMesh(core_axis_name="c", subcore_axis_name="s",
                                  num_cores=NC, num_subcores=NS)
    return pl.kernel(
        _encode_body,
        out_type=jax.ShapeDtypeStruct((2 * N_LEVELS, N_POINTS), jnp.float32),
        mesh=mesh,
        scratch_types=[
            pltpu.VMEM((2, 3, P), jnp.float32),
            pltpu.VMEM((2, G, 128), jnp.int32),
            pltpu.VMEM((2, 8 * P), jnp.int32),
            pltpu.VMEM((2, 8 * P), jnp.float32),
            pltpu.VMEM((2, 8 * P, 8), jnp.float32),
            pltpu.VMEM((2, 2, P), jnp.float32),
            pltpu.VMEM((16,), jnp.float32),
            pltpu.VMEM_SHARED((LVL, 8), jnp.float32),
            pltpu.SemaphoreType.DMA,
            pltpu.SemaphoreType.DMA,
            pltpu.SemaphoreType.DMA,
            pltpu.SemaphoreType.DMA,
            pltpu.SemaphoreType.DMA,
            pltpu.SemaphoreType.DMA,
        ],
        compiler_params=pltpu.CompilerParams(needs_layout_passes=False,
                                             use_tc_tiling_on_sc=False),
    )


def _tr_body(i_ref, p_ref, o_ref):
    a = i_ref[...]
    a = a.reshape(a.shape[0] // 2, 256)
    b = lax.dot_general(a, p_ref[...], (((1,), (0,)), ((), ())),
                        preferred_element_type=jnp.float32,
                        precision=lax.Precision.HIGHEST)
    o_ref[...] = b.reshape(a.shape[0] * 2, 128)


BT = 8192
TR_ROWS = N_LEVELS * T // 64

_PERM = np.zeros((256, 256), dtype=np.float32)
for _j in range(256):
    _PERM[(_j % 2) * 128 + _j // 2, _j] = 1.0


def _transpose_tables(gtab):
    return pl.pallas_call(
        _tr_body,
        grid=(TR_ROWS // BT,),
        in_specs=[pl.BlockSpec((BT, 128), lambda i: (i, 0)),
                  pl.BlockSpec((256, 256), lambda i: (0, 0))],
        out_specs=pl.BlockSpec((BT, 128), lambda i: (i, 0)),
        out_shape=jax.ShapeDtypeStruct((TR_ROWS, 128), jnp.float32),
    )(gtab, jnp.asarray(_PERM))


def _mlp_body(f_ref, w1_ref, b1_ref, w2_ref, b2_ref, w3_ref, b3_ref, o_ref):
    a3 = f_ref[...]
    a = a3.reshape(2 * N_LEVELS, PB)
    h = lax.dot_general(w1_ref[...], a, (((0,), (0,)), ((), ())),
                        preferred_element_type=jnp.float32)
    h = jnp.maximum(h + b1_ref[...], 0.0)
    h = lax.dot_general(w2_ref[...], h, (((0,), (0,)), ((), ())),
                        preferred_element_type=jnp.float32)
    h = jnp.maximum(h + b2_ref[...], 0.0)
    o = lax.dot_general(w3_ref[...], h, (((0,), (0,)), ((), ())),
                        preferred_element_type=jnp.float32)
    o_ref[...] = o + b3_ref[...]


PB = 8192


def _mlp(feats3, W1, b1, W2, b2, W3, b3):
    return pl.pallas_call(
        _mlp_body,
        grid=(N_POINTS // PB,),
        in_specs=[
            pl.BlockSpec((2 * N_LEVELS, PB // 128, 128), lambda i: (0, i, 0)),
            pl.BlockSpec((2 * N_LEVELS, 64), lambda i: (0, 0)),
            pl.BlockSpec((64, 1), lambda i: (0, 0)),
            pl.BlockSpec((64, 64), lambda i: (0, 0)),
            pl.BlockSpec((64, 1), lambda i: (0, 0)),
            pl.BlockSpec((64, OUT_DIM), lambda i: (0, 0)),
            pl.BlockSpec((OUT_DIM, 1), lambda i: (0, 0)),
        ],
        out_specs=pl.BlockSpec((OUT_DIM, PB), lambda i: (0, i)),
        out_shape=jax.ShapeDtypeStruct((OUT_DIM, N_POINTS), jnp.float32),
    )(feats3, W1, b1.reshape(64, 1), W2, b2.reshape(64, 1),
      W3, b3.reshape(OUT_DIM, 1))


@jax.jit
def _impl(x, tables, W1, b1, W2, b2, W3, b3):
    xt = x.T
    gtab = tables.reshape(N_LEVELS, T // 128, 128, 2)
    gtab = gtab.transpose(0, 1, 3, 2).reshape(TR_ROWS, 128)
    tab8 = _transpose_tables(gtab).reshape(N_LEVELS * T // 4, 4 * F)
    res = jnp.asarray(np.array(RES, np.float32))
    feats = _make_encode()(xt, tab8, res)
    feats3 = feats.reshape(2 * N_LEVELS, N_POINTS // 128, 128)
    out16 = _mlp(feats3, W1, b1, W2, b2, W3, b3)
    return out16.T


def kernel(x, tables, W1, b1, W2, b2, W3, b3):
    return _impl(x, tables, W1, b1, W2, b2, W3, b3)


# speedup vs baseline: 1.5385x; 1.0152x over previous
import functools

import numpy as np
import jax
import jax.numpy as jnp
from jax import lax
from jax.experimental import pallas as pl
from jax.experimental.pallas import tpu as pltpu
from jax.experimental.pallas import tpu_sc as plsc

N_LEVELS = 16
F = 2
LOG2_T = 19
T = 1 << LOG2_T
BASE = 16
FINEST = 512
OUT_DIM = 16
N_POINTS = 524288

_b = np.exp((np.log(FINEST) - np.log(BASE)) / (N_LEVELS - 1))
RES = [int(np.floor(BASE * (_b ** l))) for l in range(N_LEVELS)]
P1 = int(np.uint32(2654435761).astype(np.int32))
P2 = int(np.uint32(805459861).astype(np.int32))

NC, NS = 2, 16
NW = NC * NS
PTS = N_POINTS // NW
P = 256
G = P // 16
NCHUNK = PTS // P


LVL = T // 4


def _run_level(l, base0, tab_ref, x_hbm, feats_hbm, xr_v, idx_v, lo_v, w_v,
               rows_v, fb_v, res_v, semg, semx, semf, iot, zeros16):
    lb = zeros16 + l
    r_b = plsc.load_gather(res_v, [lb])

    pltpu.async_copy(x_hbm.at[:, pl.ds(base0, P)], xr_v.at[0], semx[0])

    def phase1(j, b):
        @pl.when(j + 1 < NCHUNK)
        def _():
            pltpu.async_copy(x_hbm.at[:, pl.ds(base0 + (j + 1) * P, P)],
                             xr_v.at[1 - b], semx[1 - b])
        pltpu.make_async_copy(x_hbm.at[:, pl.ds(base0, P)],
                              xr_v.at[b], semx[b]).wait()

        def grp_body(g, c3):
            o = g * 16

            def nrm(d):
                v = xr_v[b, d, pl.ds(o, 16)]
                v = jnp.minimum(jnp.maximum(v * 0.5 + 0.5, 0.0), 1.0)
                return v * r_b
            pos0 = nrm(0)
            pos1 = nrm(1)
            pos2 = nrm(2)
            i0 = pos0.astype(jnp.int32)
            i1 = pos1.astype(jnp.int32)
            i2 = pos2.astype(jnp.int32)
            w0 = pos0 - i0.astype(jnp.float32)
            w1 = pos1 - i1.astype(jnp.float32)
            w2 = pos2 - i2.astype(jnp.float32)
            hx = (i0, i0 + 1)
            hy0 = i1 * P1
            hy = (hy0, hy0 + P1)
            hz0 = i2 * P2
            hz = (hz0, hz0 + P2)
            u0 = 1.0 - w0
            u1 = 1.0 - w1
            u2 = 1.0 - w2
            wx = (u0, w0)
            cyz = (u1 * u2, u1 * w2, w1 * u2, w1 * w2)
            eyz = (hy[0] ^ hz[0], hy[0] ^ hz[1], hy[1] ^ hz[0], hy[1] ^ hz[1])
            for c in range(8):
                bi, bj, bk = (c >> 2) & 1, (c >> 1) & 1, c & 1
                h = hx[bi] ^ eyz[bj * 2 + bk]
                idxc = h & (T - 1)
                wc = wx[bi] * cyz[bj * 2 + bk]
                idx_v[b, g, pl.ds(c * 16, 16)] = lax.shift_right_logical(idxc, 2)
                lo_v[b, pl.ds(g * 128 + c * 16, 16)] = (idxc & 3) * 2
                w_v[b, pl.ds(g * 128 + c * 16, 16)] = wc
            pltpu.async_copy(tab_ref.at[idx_v.at[b].at[g]],
                             rows_v.at[b].at[pl.ds(g * 128, 128)], semg[b])
            return c3

        @plsc.parallel_loop(0, G, unroll=2)
        def _(g):
            grp_body(g, 0)

    def phase2(q, b):
        @pl.when(q >= 2)
        def _():
            pltpu.make_async_copy(
                fb_v.at[b],
                feats_hbm.at[pl.ds(2 * l, 2), pl.ds(base0, P)],
                semf[b]).wait()

        def acc_body(g, c3):
            pltpu.make_async_copy(tab_ref.at[idx_v.at[b].at[g]],
                                  rows_v.at[b].at[pl.ds(g * 128, 128)],
                                  semg[b]).wait()
            acc0 = jnp.zeros((16,), jnp.float32)
            acc1 = jnp.zeros((16,), jnp.float32)
            for c in range(8):
                lanes = g * 128 + c * 16 + iot
                lo = lo_v[b, pl.ds(g * 128 + c * 16, 16)]
                f0 = plsc.load_gather(rows_v.at[b], [lanes, lo])
                f1 = plsc.load_gather(rows_v.at[b], [lanes, lo + 1])
                wc = w_v[b, pl.ds(g * 128 + c * 16, 16)]
                acc0 = acc0 + f0 * wc
                acc1 = acc1 + f1 * wc
            fb_v[b, 0, pl.ds(g * 16, 16)] = acc0
            fb_v[b, 1, pl.ds(g * 16, 16)] = acc1
            return c3

        @plsc.parallel_loop(0, G, unroll=2)
        def _(g):
            acc_body(g, 0)
        pltpu.async_copy(fb_v.at[b],
                         feats_hbm.at[pl.ds(2 * l, 2),
                                      pl.ds(base0 + q * P, P)], semf[b])

    def loop_body(j, carry):
        for par in (0, 1):
            @pl.when(jnp.logical_and(j < NCHUNK, j % 2 == par))
            def _(par=par):
                phase1(j, par)
        q = j - 1
        for par in (0, 1):
            @pl.when(jnp.logical_and(j > 0, q % 2 == par))
            def _(par=par):
                phase2(q, par)
        return carry
    lax.fori_loop(0, NCHUNK + 1, loop_body, 0)

    for par in (0, 1):
        pltpu.make_async_copy(
            fb_v.at[par],
            feats_hbm.at[pl.ds(2 * l, 2), pl.ds(base0, P)], semf[par]).wait()


def _encode_body(x_hbm, tab_hbm, res_hbm, feats_hbm,
                 xr_v, idx_v, lo_v, w_v, rows_v, fb_v, res_v,
                 tabA, semg0, semg1, semx0, semx1, semf0, semf1):
    cid = lax.axis_index("c")
    sid = lax.axis_index("s")
    wid = sid * NC + cid
    base0 = wid * PTS
    pltpu.sync_copy(res_hbm, res_v)
    iot = lax.iota(jnp.int32, 16)
    zeros16 = jnp.zeros((16,), jnp.int32)

    SL = LVL // NS

    def level_body(l, carry):
        pltpu.sync_copy(tab_hbm.at[pl.ds(l * LVL + sid * SL, SL)],
                        tabA.at[pl.ds(sid * SL, SL)])
        plsc.subcore_barrier()
        _run_level(l, base0, tabA, x_hbm, feats_hbm, xr_v, idx_v, lo_v,
                   w_v, rows_v, fb_v, res_v, (semg0, semg1),
                   (semx0, semx1), (semf0, semf1), iot, zeros16)
        plsc.subcore_barrier()
        return carry
    lax.fori_loop(0, N_LEVELS, level_body, 0)


def _make_encode():
    mesh = plsc.VectorSubcoreMesh(core_axis_name="c", subcore_axis_name="s",
                                  num_cores=NC, num_subcores=NS)
    return pl.kernel(
        _encode_body,
        out_type=jax.ShapeDtypeStruct((2 * N_LEVELS, N_POINTS), jnp.float32),
        mesh=mesh,
        scratch_types=[
            pltpu.VMEM((2, 3, P), jnp.float32),
            pltpu.VMEM((2, G, 128), jnp.int32),
            pltpu.VMEM((2, 8 * P), jnp.int32),
            pltpu.VMEM((2, 8 * P), jnp.float32),
            pltpu.VMEM((2, 8 * P, 8), jnp.float32),
            pltpu.VMEM((2, 2, P), jnp.float32),
            pltpu.VMEM((16,), jnp.float32),
            pltpu.VMEM_SHARED((LVL, 8), jnp.float32),
            pltpu.SemaphoreType.DMA,
            pltpu.SemaphoreType.DMA,
            pltpu.SemaphoreType.DMA,
            pltpu.SemaphoreType.DMA,
            pltpu.SemaphoreType.DMA,
            pltpu.SemaphoreType.DMA,
        ],
        compiler_params=pltpu.CompilerParams(needs_layout_passes=False,
                                             use_tc_tiling_on_sc=False),
    )


def _tr_body(i_ref, p_ref, o_ref):
    a = i_ref[...]
    a = a.reshape(a.shape[0] // 2, 256)
    b = lax.dot_general(a, p_ref[...], (((1,), (0,)), ((), ())),
                        preferred_element_type=jnp.float32,
                        precision=lax.Precision.HIGHEST)
    o_ref[...] = b.reshape(a.shape[0] * 2, 128)


BT = 8192
TR_ROWS = N_LEVELS * T // 64

_PERM = np.zeros((256, 256), dtype=np.float32)
for _j in range(256):
    _PERM[(_j % 2) * 128 + _j // 2, _j] = 1.0


def _transpose_tables(gtab):
    return pl.pallas_call(
        _tr_body,
        grid=(TR_ROWS // BT,),
        in_specs=[pl.BlockSpec((BT, 128), lambda i: (i, 0)),
                  pl.BlockSpec((256, 256), lambda i: (0, 0))],
        out_specs=pl.BlockSpec((BT, 128), lambda i: (i, 0)),
        out_shape=jax.ShapeDtypeStruct((TR_ROWS, 128), jnp.float32),
    )(gtab, jnp.asarray(_PERM))


def _mlp_body(f_ref, w1_ref, b1_ref, w2_ref, b2_ref, w3_ref, b3_ref, o_ref):
    a3 = f_ref[...]
    a = a3.reshape(2 * N_LEVELS, PB)
    h = lax.dot_general(w1_ref[...], a, (((0,), (0,)), ((), ())),
                        preferred_element_type=jnp.float32)
    h = jnp.maximum(h + b1_ref[...], 0.0)
    h = lax.dot_general(w2_ref[...], h, (((0,), (0,)), ((), ())),
                        preferred_element_type=jnp.float32)
    h = jnp.maximum(h + b2_ref[...], 0.0)
    o = lax.dot_general(w3_ref[...], h, (((0,), (0,)), ((), ())),
                        preferred_element_type=jnp.float32)
    o_ref[...] = o + b3_ref[...]


PB = 16384


def _mlp(feats3, W1, b1, W2, b2, W3, b3):
    return pl.pallas_call(
        _mlp_body,
        grid=(N_POINTS // PB,),
        in_specs=[
            pl.BlockSpec((2 * N_LEVELS, PB // 128, 128), lambda i: (0, i, 0)),
            pl.BlockSpec((2 * N_LEVELS, 64), lambda i: (0, 0)),
            pl.BlockSpec((64, 1), lambda i: (0, 0)),
            pl.BlockSpec((64, 64), lambda i: (0, 0)),
            pl.BlockSpec((64, 1), lambda i: (0, 0)),
            pl.BlockSpec((64, OUT_DIM), lambda i: (0, 0)),
            pl.BlockSpec((OUT_DIM, 1), lambda i: (0, 0)),
        ],
        out_specs=pl.BlockSpec((OUT_DIM, PB), lambda i: (0, i)),
        out_shape=jax.ShapeDtypeStruct((OUT_DIM, N_POINTS), jnp.float32),
    )(feats3, W1, b1.reshape(64, 1), W2, b2.reshape(64, 1),
      W3, b3.reshape(OUT_DIM, 1))


@jax.jit
def _impl(x, tables, W1, b1, W2, b2, W3, b3):
    xt = x.T
    gtab = tables.reshape(N_LEVELS, T // 128, 128, 2)
    gtab = gtab.transpose(0, 1, 3, 2).reshape(TR_ROWS, 128)
    tab8 = _transpose_tables(gtab).reshape(N_LEVELS * T // 4, 4 * F)
    res = jnp.asarray(np.array(RES, np.float32))
    feats = _make_encode()(xt, tab8, res)
    feats3 = feats.reshape(2 * N_LEVELS, N_POINTS // 128, 128)
    out16 = _mlp(feats3, W1, b1, W2, b2, W3, b3)
    return out16.T


def kernel(x, tables, W1, b1, W2, b2, W3, b3):
    return _impl(x, tables, W1, b1, W2, b2, W3, b3)


# speedup vs baseline: 1.5503x; 1.0077x over previous
import functools

import numpy as np
import jax
import jax.numpy as jnp
from jax import lax
from jax.experimental import pallas as pl
from jax.experimental.pallas import tpu as pltpu
from jax.experimental.pallas import tpu_sc as plsc

N_LEVELS = 16
F = 2
LOG2_T = 19
T = 1 << LOG2_T
BASE = 16
FINEST = 512
OUT_DIM = 16
N_POINTS = 524288

_b = np.exp((np.log(FINEST) - np.log(BASE)) / (N_LEVELS - 1))
RES = [int(np.floor(BASE * (_b ** l))) for l in range(N_LEVELS)]
P1 = int(np.uint32(2654435761).astype(np.int32))
P2 = int(np.uint32(805459861).astype(np.int32))

NC, NS = 2, 16
NW = NC * NS
PTS = N_POINTS // NW
P = 256
G = P // 16
NCHUNK = PTS // P


LVL = T // 4


def _run_level(l, base0, tab_ref, x_hbm, feats_hbm, xr_v, idx_v, lo_v, w_v,
               rows_v, fb_v, res_v, semg, semx, semf, iot, zeros16):
    lb = zeros16 + l
    r_b = plsc.load_gather(res_v, [lb])

    pltpu.async_copy(x_hbm.at[:, pl.ds(base0, P)], xr_v.at[0], semx[0])

    def phase1(j, b):
        @pl.when(j + 1 < NCHUNK)
        def _():
            pltpu.async_copy(x_hbm.at[:, pl.ds(base0 + (j + 1) * P, P)],
                             xr_v.at[1 - b], semx[1 - b])
        pltpu.make_async_copy(x_hbm.at[:, pl.ds(base0, P)],
                              xr_v.at[b], semx[b]).wait()

        def grp_body(g, c3):
            o = g * 16

            def nrm(d):
                v = xr_v[b, d, pl.ds(o, 16)]
                v = jnp.minimum(jnp.maximum(v * 0.5 + 0.5, 0.0), 1.0)
                return v * r_b
            pos0 = nrm(0)
            pos1 = nrm(1)
            pos2 = nrm(2)
            i0 = pos0.astype(jnp.int32)
            i1 = pos1.astype(jnp.int32)
            i2 = pos2.astype(jnp.int32)
            w0 = pos0 - i0.astype(jnp.float32)
            w1 = pos1 - i1.astype(jnp.float32)
            w2 = pos2 - i2.astype(jnp.float32)
            hx = (i0, i0 + 1)
            hy0 = i1 * P1
            hy = (hy0, hy0 + P1)
            hz0 = i2 * P2
            hz = (hz0, hz0 + P2)
            u0 = 1.0 - w0
            u1 = 1.0 - w1
            u2 = 1.0 - w2
            wx = (u0, w0)
            cyz = (u1 * u2, u1 * w2, w1 * u2, w1 * w2)
            eyz = (hy[0] ^ hz[0], hy[0] ^ hz[1], hy[1] ^ hz[0], hy[1] ^ hz[1])
            for c in range(8):
                bi, bj, bk = (c >> 2) & 1, (c >> 1) & 1, c & 1
                h = hx[bi] ^ eyz[bj * 2 + bk]
                idxc = h & (T - 1)
                wc = wx[bi] * cyz[bj * 2 + bk]
                idx_v[b, g, pl.ds(c * 16, 16)] = lax.shift_right_logical(idxc, 2)
                lo_v[b, pl.ds(g * 128 + c * 16, 16)] = (idxc & 3) * 2
                w_v[b, pl.ds(g * 128 + c * 16, 16)] = wc
            pltpu.async_copy(tab_ref.at[idx_v.at[b].at[g]],
                             rows_v.at[b].at[pl.ds(g * 128, 128)], semg[b])
            return c3

        @plsc.parallel_loop(0, G, unroll=2)
        def _(g):
            grp_body(g, 0)

    def phase2(q, b):
        @pl.when(q >= 2)
        def _():
            pltpu.make_async_copy(
                fb_v.at[b],
                feats_hbm.at[pl.ds(2 * l, 2), pl.ds(base0, P)],
                semf[b]).wait()

        def acc_body(g, c3):
            pltpu.make_async_copy(tab_ref.at[idx_v.at[b].at[g]],
                                  rows_v.at[b].at[pl.ds(g * 128, 128)],
                                  semg[b]).wait()
            acc0 = jnp.zeros((16,), jnp.float32)
            acc1 = jnp.zeros((16,), jnp.float32)
            for c in range(8):
                lanes = g * 128 + c * 16 + iot
                lo = lo_v[b, pl.ds(g * 128 + c * 16, 16)]
                f0 = plsc.load_gather(rows_v.at[b], [lanes, lo])
                f1 = plsc.load_gather(rows_v.at[b], [lanes, lo + 1])
                wc = w_v[b, pl.ds(g * 128 + c * 16, 16)]
                acc0 = acc0 + f0 * wc
                acc1 = acc1 + f1 * wc
            fb_v[b, 0, pl.ds(g * 16, 16)] = acc0
            fb_v[b, 1, pl.ds(g * 16, 16)] = acc1
            return c3

        @plsc.parallel_loop(0, G, unroll=2)
        def _(g):
            acc_body(g, 0)
        pltpu.async_copy(fb_v.at[b],
                         feats_hbm.at[pl.ds(2 * l, 2),
                                      pl.ds(base0 + q * P, P)], semf[b])

    def loop_body(j, carry):
        for par in (0, 1):
            @pl.when(jnp.logical_and(j < NCHUNK, j % 2 == par))
            def _(par=par):
                phase1(j, par)
        q = j - 1
        for par in (0, 1):
            @pl.when(jnp.logical_and(j > 0, q % 2 == par))
            def _(par=par):
                phase2(q, par)
        return carry
    lax.fori_loop(0, NCHUNK + 1, loop_body, 0)

    for par in (0, 1):
        pltpu.make_async_copy(
            fb_v.at[par],
            feats_hbm.at[pl.ds(2 * l, 2), pl.ds(base0, P)], semf[par]).wait()


def _encode_body(x_hbm, tab_hbm, res_hbm, feats_hbm,
                 xr_v, idx_v, lo_v, w_v, rows_v, fb_v, res_v,
                 tabA, semg0, semg1, semx0, semx1, semf0, semf1):
    cid = lax.axis_index("c")
    sid = lax.axis_index("s")
    wid = sid * NC + cid
    base0 = wid * PTS
    pltpu.sync_copy(res_hbm, res_v)
    iot = lax.iota(jnp.int32, 16)
    zeros16 = jnp.zeros((16,), jnp.int32)

    SL = LVL // NS

    def level_body(l, carry):
        pltpu.sync_copy(tab_hbm.at[pl.ds(l * LVL + sid * SL, SL)],
                        tabA.at[pl.ds(sid * SL, SL)])
        plsc.subcore_barrier()
        _run_level(l, base0, tabA, x_hbm, feats_hbm, xr_v, idx_v, lo_v,
                   w_v, rows_v, fb_v, res_v, (semg0, semg1),
                   (semx0, semx1), (semf0, semf1), iot, zeros16)
        plsc.subcore_barrier()
        return carry
    lax.fori_loop(0, N_LEVELS, level_body, 0)


def _make_encode():
    mesh = plsc.VectorSubcoreMesh(core_axis_name="c", subcore_axis_name="s",
                                  num_cores=NC, num_subcores=NS)
    return pl.kernel(
        _encode_body,
        out_type=jax.ShapeDtypeStruct((2 * N_LEVELS, N_POINTS), jnp.float32),
        mesh=mesh,
        scratch_types=[
            pltpu.VMEM((2, 3, P), jnp.float32),
            pltpu.VMEM((2, G, 128), jnp.int32),
            pltpu.VMEM((2, 8 * P), jnp.int32),
            pltpu.VMEM((2, 8 * P), jnp.float32),
            pltpu.VMEM((2, 8 * P, 8), jnp.float32),
            pltpu.VMEM((2, 2, P), jnp.float32),
            pltpu.VMEM((16,), jnp.float32),
            pltpu.VMEM_SHARED((LVL, 8), jnp.float32),
            pltpu.SemaphoreType.DMA,
            pltpu.SemaphoreType.DMA,
            pltpu.SemaphoreType.DMA,
            pltpu.SemaphoreType.DMA,
            pltpu.SemaphoreType.DMA,
            pltpu.SemaphoreType.DMA,
        ],
        compiler_params=pltpu.CompilerParams(needs_layout_passes=False,
                                             use_tc_tiling_on_sc=False),
    )


def _tr_body(i_ref, p_ref, o_ref):
    a = i_ref[...]
    a = a.reshape(a.shape[0] // 2, 256)
    b = lax.dot_general(a, p_ref[...], (((1,), (0,)), ((), ())),
                        preferred_element_type=jnp.float32,
                        precision=lax.Precision.HIGHEST)
    o_ref[...] = b.reshape(a.shape[0] * 2, 128)


BT = 8192
TR_ROWS = N_LEVELS * T // 64

_PERM = np.zeros((256, 256), dtype=np.float32)
for _j in range(256):
    _PERM[(_j % 2) * 128 + _j // 2, _j] = 1.0


def _transpose_tables(gtab):
    return pl.pallas_call(
        _tr_body,
        grid=(TR_ROWS // BT,),
        in_specs=[pl.BlockSpec((BT, 128), lambda i: (i, 0)),
                  pl.BlockSpec((256, 256), lambda i: (0, 0))],
        out_specs=pl.BlockSpec((BT, 128), lambda i: (i, 0)),
        out_shape=jax.ShapeDtypeStruct((TR_ROWS, 128), jnp.float32),
    )(gtab, jnp.asarray(_PERM))


def _mlp_body(f_ref, w1_ref, b1_ref, w2_ref, b2_ref, w3_ref, b3_ref, o_ref):
    a3 = f_ref[...]
    a = a3.reshape(2 * N_LEVELS, PB)
    h = lax.dot_general(w1_ref[...], a, (((0,), (0,)), ((), ())),
                        preferred_element_type=jnp.float32)
    h = jnp.maximum(h + b1_ref[...], 0.0)
    h = lax.dot_general(w2_ref[...], h, (((0,), (0,)), ((), ())),
                        preferred_element_type=jnp.float32)
    h = jnp.maximum(h + b2_ref[...], 0.0)
    o = lax.dot_general(w3_ref[...], h, (((0,), (0,)), ((), ())),
                        preferred_element_type=jnp.float32)
    o_ref[...] = o + b3_ref[...]


PB = 32768


def _mlp(feats3, W1, b1, W2, b2, W3, b3):
    return pl.pallas_call(
        _mlp_body,
        grid=(N_POINTS // PB,),
        in_specs=[
            pl.BlockSpec((2 * N_LEVELS, PB // 128, 128), lambda i: (0, i, 0)),
            pl.BlockSpec((2 * N_LEVELS, 64), lambda i: (0, 0)),
            pl.BlockSpec((64, 1), lambda i: (0, 0)),
            pl.BlockSpec((64, 64), lambda i: (0, 0)),
            pl.BlockSpec((64, 1), lambda i: (0, 0)),
            pl.BlockSpec((64, OUT_DIM), lambda i: (0, 0)),
            pl.BlockSpec((OUT_DIM, 1), lambda i: (0, 0)),
        ],
        out_specs=pl.BlockSpec((OUT_DIM, PB), lambda i: (0, i)),
        out_shape=jax.ShapeDtypeStruct((OUT_DIM, N_POINTS), jnp.float32),
    )(feats3, W1, b1.reshape(64, 1), W2, b2.reshape(64, 1),
      W3, b3.reshape(OUT_DIM, 1))


@jax.jit
def _impl(x, tables, W1, b1, W2, b2, W3, b3):
    xt = x.T
    gtab = tables.reshape(N_LEVELS, T // 128, 128, 2)
    gtab = gtab.transpose(0, 1, 3, 2).reshape(TR_ROWS, 128)
    tab8 = _transpose_tables(gtab).reshape(N_LEVELS * T // 4, 4 * F)
    res = jnp.asarray(np.array(RES, np.float32))
    feats = _make_encode()(xt, tab8, res)
    feats3 = feats.reshape(2 * N_LEVELS, N_POINTS // 128, 128)
    out16 = _mlp(feats3, W1, b1, W2, b2, W3, b3)
    return out16.T


def kernel(x, tables, W1, b1, W2, b2, W3, b3):
    return _impl(x, tables, W1, b1, W2, b2, W3, b3)
